# baseline (speedup 1.0000x reference)
import jax
import jax.numpy as jnp
from jax.experimental import pallas as pl
from jax.experimental.pallas import tpu as pltpu

N_EXPERTS = 64
K = 8
BT = 4096
BC = 1024


def _router_block(xa_ref, xb_ref, wt_ref, b_ref, p_ref, m_ref):
    wt = wt_ref[...]
    b_col = b_ref[...]
    eye = jnp.eye(N_EXPERTS, dtype=jnp.float32)
    ones_col = jnp.ones((N_EXPERTS, 1), dtype=jnp.float32)
    one11 = jnp.ones((1, 1), dtype=jnp.float32)
    half = BT // (2 * BC)
    for c in range(BT // BC):
        sl = slice(c * BC, (c + 1) * BC)
        x_ref = xa_ref if c < half else xb_ref
        isl = slice((c % half) * BC, (c % half + 1) * BC)
        lt = jax.lax.dot_general(
            wt, x_ref[isl, :],
            dimension_numbers=(((0,), (1,)), ((), ())),
            preferred_element_type=jnp.float32,
        ) + b_col
        w = lt
        for _ in range(K - 1):
            m = jnp.max(w, axis=0, keepdims=True)
            w = jnp.where(w == m, -jnp.inf, w)
        t_row = jnp.max(w, axis=0, keepdims=True)
        ln = jax.lax.dot_general(
            lt, eye,
            dimension_numbers=(((0,), (0,)), ((), ())),
            preferred_element_type=jnp.float32,
        )
        t_col = jax.lax.dot_general(
            t_row, one11,
            dimension_numbers=(((0,), (0,)), ((), ())),
            preferred_element_type=jnp.float32,
        )
        e = jnp.exp(ln)
        s_col = jnp.dot(e, ones_col, preferred_element_type=jnp.float32)
        p_ref[sl, :] = e / s_col
        m_ref[sl, :] = ln >= t_col


@jax.jit
def kernel(x, W, b):
    n_tokens, d_model = x.shape
    wt = W.T
    b_col = b.reshape(N_EXPERTS, 1)
    probs, mask = pl.pallas_call(
        _router_block,
        grid=(n_tokens // BT,),
        in_specs=[
            pl.BlockSpec((BT // 2, d_model), lambda i: (2 * i, 0)),
            pl.BlockSpec((BT // 2, d_model), lambda i: (2 * i + 1, 0)),
            pl.BlockSpec((d_model, N_EXPERTS), lambda i: (0, 0)),
            pl.BlockSpec((N_EXPERTS, 1), lambda i: (0, 0)),
        ],
        out_specs=[
            pl.BlockSpec((BT, N_EXPERTS), lambda i: (i, 0)),
            pl.BlockSpec((BT, N_EXPERTS), lambda i: (i, 0)),
        ],
        out_shape=[
            jax.ShapeDtypeStruct((n_tokens, N_EXPERTS), jnp.float32),
            jax.ShapeDtypeStruct((n_tokens, N_EXPERTS), jnp.bool_),
        ],
        compiler_params=pltpu.CompilerParams(
            dimension_semantics=("parallel",),
        ),
    )(x, x, wt, b_col)
    return (probs, mask)


# speedup vs baseline: 1.0545x; 1.0545x over previous
import jax
import jax.numpy as jnp
from jax.experimental import pallas as pl
from jax.experimental.pallas import tpu as pltpu

N_EXPERTS = 64
K = 8
BT = 4096
BC = 1024


def _router_block(xa_ref, xb_ref, wt_ref, b_ref, p_ref, m_ref):
    wt = wt_ref[...]
    b_col = b_ref[...]
    half = BT // (2 * BC)
    for c in range(BT // BC):
        sl = slice(c * BC, (c + 1) * BC)
        x_ref = xa_ref if c < half else xb_ref
        isl = slice((c % half) * BC, (c % half + 1) * BC)
        lt = jax.lax.dot_general(
            wt, x_ref[isl, :],
            dimension_numbers=(((0,), (1,)), ((), ())),
            preferred_element_type=jnp.float32,
        ) + b_col
        e = jnp.exp(lt)
        s = jnp.sum(e, axis=0, keepdims=True)
        w = lt
        for _ in range(K - 1):
            m = jnp.max(w, axis=0, keepdims=True)
            w = jnp.where(w == m, -jnp.inf, w)
        t = jnp.max(w, axis=0, keepdims=True)
        p_ref[sl, :] = (e / s).T
        m_ref[sl, :] = (lt >= t).T


@jax.jit
def kernel(x, W, b):
    n_tokens, d_model = x.shape
    wt = W.T
    b_col = b.reshape(N_EXPERTS, 1)
    probs, mask = pl.pallas_call(
        _router_block,
        grid=(n_tokens // BT,),
        in_specs=[
            pl.BlockSpec((BT // 2, d_model), lambda i: (2 * i, 0)),
            pl.BlockSpec((BT // 2, d_model), lambda i: (2 * i + 1, 0)),
            pl.BlockSpec((d_model, N_EXPERTS), lambda i: (0, 0)),
            pl.BlockSpec((N_EXPERTS, 1), lambda i: (0, 0)),
        ],
        out_specs=[
            pl.BlockSpec((BT, N_EXPERTS), lambda i: (i, 0)),
            pl.BlockSpec((BT, N_EXPERTS), lambda i: (i, 0)),
        ],
        out_shape=[
            jax.ShapeDtypeStruct((n_tokens, N_EXPERTS), jnp.float32),
            jax.ShapeDtypeStruct((n_tokens, N_EXPERTS), jnp.bool_),
        ],
        compiler_params=pltpu.CompilerParams(
            dimension_semantics=("parallel",),
        ),
    )(x, x, wt, b_col)
    return (probs, mask)
